# Initial kernel scaffold; baseline (speedup 1.0000x reference)
import jax
import jax.numpy as jnp
from jax.experimental import pallas as pl


def kernel(texts, embedding):
    raise NotImplementedError("write your pallas kernel here")


# speedup vs baseline: 6.2954x; 6.2954x over previous
import functools

import jax
import jax.numpy as jnp
from jax import lax
from jax.experimental import pallas as pl
from jax.experimental.pallas import tpu as pltpu
from jax.experimental.pallas import tpu_sc as plsc

VOCAB = 100000
DIM = 128
BATCH = 4096
SEQ = 200
CHUNK = 100
NCHUNK = SEQ // CHUNK
NC = 2
NS = 16
NW = NC * NS
BPW = BATCH // NW
CPW = BPW * NCHUNK
NLANE = 16
NVEC = DIM // NLANE


def _make_kernel():
    mesh = plsc.VectorSubcoreMesh(core_axis_name="c", subcore_axis_name="s")

    @functools.partial(
        pl.kernel,
        out_type=jax.ShapeDtypeStruct((BATCH, DIM), jnp.float32),
        mesh=mesh,
        scratch_types=[
            pltpu.VMEM((CPW, CHUNK), jnp.int32),
            pltpu.VMEM((2, CHUNK, DIM), jnp.float32),
            pltpu.VMEM((BPW, DIM), jnp.float32),
            pltpu.SemaphoreType.DMA,
            pltpu.SemaphoreType.DMA,
        ],
    )
    def enc(texts_hbm, emb_hbm, out_hbm, idx_v, rows_v, out_v, sem0, sem1):
        wid = lax.axis_index("s") * NC + lax.axis_index("c")
        pltpu.sync_copy(texts_hbm.at[pl.ds(wid * CPW, CPW)], idx_v)
        sems = (sem0, sem1)

        def acc_chunk(p, ci):
            pltpu.async_copy(emb_hbm.at[idx_v.at[ci]], rows_v.at[p], sems[p]).wait()

            def jbody(j, accs):
                return tuple(
                    accs[g] + rows_v[p, j, pl.ds(NLANE * g, NLANE)]
                    for g in range(NVEC)
                )

            zeros = tuple(jnp.zeros((NLANE,), jnp.float32) for _ in range(NVEC))
            return lax.fori_loop(0, CHUNK, jbody, zeros)

        def row_body(t, carry):
            a = acc_chunk(0, NCHUNK * t)
            b = acc_chunk(1, NCHUNK * t + 1)
            scale = jnp.float32(1.0 / SEQ)
            for g in range(NVEC):
                out_v[t, pl.ds(NLANE * g, NLANE)] = (a[g] + b[g]) * scale
            return carry

        lax.fori_loop(0, BPW, row_body, 0)
        pltpu.sync_copy(out_v, out_hbm.at[pl.ds(wid * BPW, BPW)])

    def kern(texts, embedding):
        texts_r = texts.reshape(BATCH * NCHUNK, CHUNK)
        return enc(texts_r, embedding)

    return kern


kernel = _make_kernel()


# speedup vs baseline: 16.8701x; 2.6798x over previous
import functools

import jax
import jax.numpy as jnp
from jax import lax
from jax.experimental import pallas as pl
from jax.experimental.pallas import tpu as pltpu
from jax.experimental.pallas import tpu_sc as plsc

VOCAB = 100000
DIM = 128
BATCH = 4096
SEQ = 200
CHUNK = 100
NCHUNK = SEQ // CHUNK
NC = 2
NS = 16
NW = NC * NS
BPW = BATCH // NW
CPW = BPW * NCHUNK
NLANE = 16
NVEC = DIM // NLANE
NBUF = 4
UNROLL = 4


def _make_kernel():
    mesh = plsc.VectorSubcoreMesh(core_axis_name="c", subcore_axis_name="s")

    @functools.partial(
        pl.kernel,
        out_type=jax.ShapeDtypeStruct((BATCH, DIM), jnp.float32),
        mesh=mesh,
        scratch_types=[
            pltpu.VMEM((CPW, CHUNK), jnp.int32),
            pltpu.VMEM((NBUF, CHUNK, DIM), jnp.float32),
            pltpu.VMEM((BPW, DIM), jnp.float32),
            [pltpu.SemaphoreType.DMA] * NBUF,
        ],
    )
    def enc(texts_hbm, emb_hbm, out_hbm, idx_v, rows_v, out_v, sems):
        wid = lax.axis_index("s") * NC + lax.axis_index("c")
        pltpu.sync_copy(texts_hbm.at[pl.ds(wid * CPW, CPW)], idx_v)

        def start(p, ci):
            return pltpu.async_copy(
                emb_hbm.at[idx_v.at[ci]], rows_v.at[p], sems[p])

        def wait(p, ci):
            pltpu.make_async_copy(
                emb_hbm.at[idx_v.at[ci]], rows_v.at[p], sems[p]).wait()

        for p in range(NBUF):
            start(p, p)

        def acc_chunk(p, ci):
            wait(p, ci)

            def jbody(j, accs):
                accs = list(accs)
                for u in range(UNROLL):
                    for g in range(NVEC):
                        accs[g] = accs[g] + rows_v[
                            p, UNROLL * j + u, pl.ds(NLANE * g, NLANE)]
                return tuple(accs)

            zeros = tuple(jnp.zeros((NLANE,), jnp.float32) for _ in range(NVEC))
            accs = lax.fori_loop(0, CHUNK // UNROLL, jbody, zeros)

            @pl.when(ci + NBUF < CPW)
            def _():
                start(p, ci + NBUF)

            return accs

        scale = jnp.float32(1.0 / SEQ)

        def pair_body(u, carry):
            for r in range(NBUF // NCHUNK):
                t = (NBUF // NCHUNK) * u + r
                base = NBUF * u + NCHUNK * r
                a = acc_chunk(NCHUNK * r, base)
                b = acc_chunk(NCHUNK * r + 1, base + 1)
                for g in range(NVEC):
                    out_v[t, pl.ds(NLANE * g, NLANE)] = (a[g] + b[g]) * scale
            return carry

        lax.fori_loop(0, CPW // NBUF, pair_body, 0)
        pltpu.sync_copy(out_v, out_hbm.at[pl.ds(wid * BPW, BPW)])

    def kern(texts, embedding):
        texts_r = texts.reshape(BATCH * NCHUNK, CHUNK)
        return enc(texts_r, embedding)

    return kern


kernel = _make_kernel()


# speedup vs baseline: 17.0079x; 1.0082x over previous
import functools

import jax
import jax.numpy as jnp
from jax import lax
from jax.experimental import pallas as pl
from jax.experimental.pallas import tpu as pltpu
from jax.experimental.pallas import tpu_sc as plsc

VOCAB = 100000
DIM = 128
BATCH = 4096
SEQ = 200
CHUNK = 100
NCHUNK = SEQ // CHUNK
NC = 2
NS = 16
NW = NC * NS
BPW = BATCH // NW
CPW = BPW * NCHUNK
NLANE = 16
NVEC = DIM // NLANE
NBUF = 6
UNROLL = 10


def _make_kernel():
    mesh = plsc.VectorSubcoreMesh(core_axis_name="c", subcore_axis_name="s")

    @functools.partial(
        pl.kernel,
        out_type=jax.ShapeDtypeStruct((BATCH, DIM), jnp.float32),
        mesh=mesh,
        scratch_types=[
            pltpu.VMEM((CPW, CHUNK), jnp.int32),
            pltpu.VMEM((NBUF, CHUNK, DIM), jnp.float32),
            pltpu.VMEM((BPW, DIM), jnp.float32),
            [pltpu.SemaphoreType.DMA] * NBUF,
        ],
    )
    def enc(texts_hbm, emb_hbm, out_hbm, idx_v, rows_v, out_v, sems):
        wid = lax.axis_index("s") * NC + lax.axis_index("c")
        pltpu.sync_copy(texts_hbm.at[pl.ds(wid * CPW, CPW)], idx_v)

        def start(p, ci):
            return pltpu.async_copy(
                emb_hbm.at[idx_v.at[ci]], rows_v.at[p], sems[p])

        def wait(p, ci):
            pltpu.make_async_copy(
                emb_hbm.at[idx_v.at[ci]], rows_v.at[p], sems[p]).wait()

        for p in range(NBUF):
            start(p, p)

        def acc_chunk(p, ci):
            wait(p, ci)

            def jbody(j, accs):
                accs = list(accs)
                for u in range(UNROLL):
                    for g in range(NVEC):
                        accs[g] = accs[g] + rows_v[
                            p, UNROLL * j + u, pl.ds(NLANE * g, NLANE)]
                return tuple(accs)

            zeros = tuple(jnp.zeros((NLANE,), jnp.float32) for _ in range(NVEC))
            accs = lax.fori_loop(0, CHUNK // UNROLL, jbody, zeros)

            @pl.when(ci + NBUF < CPW)
            def _():
                start(p, ci + NBUF)

            return accs

        scale = jnp.float32(1.0 / SEQ)

        def pair_body(u, carry):
            for r in range(NBUF // NCHUNK):
                t = (NBUF // NCHUNK) * u + r
                base = NBUF * u + NCHUNK * r
                a = acc_chunk(NCHUNK * r, base)
                b = acc_chunk(NCHUNK * r + 1, base + 1)
                for g in range(NVEC):
                    out_v[t, pl.ds(NLANE * g, NLANE)] = (a[g] + b[g]) * scale
            return carry

        lax.fori_loop(0, CPW // NBUF, pair_body, 0)
        pltpu.sync_copy(out_v, out_hbm.at[pl.ds(wid * BPW, BPW)])

    def kern(texts, embedding):
        texts_r = texts.reshape(BATCH * NCHUNK, CHUNK)
        return enc(texts_r, embedding)

    return kern


kernel = _make_kernel()
